# baseline (speedup 1.0000x reference)
import functools
import math

import jax
import jax.numpy as jnp
from jax import lax
from jax.experimental import pallas as pl
from jax.experimental.pallas import tpu as pltpu
from jax.experimental.pallas import tpu_sc as plsc

D_MODEL = 64
SCALE = math.sqrt(D_MODEL)
LANES = 16
NC, NS = 2, 16
NW = NC * NS
W = 128
NBUF = 2


def _sc_embed(xt, lut):
    hist, batch = xt.shape
    cpw = batch // W // NW
    n_win = hist * cpw
    assert batch % (W * NW) == 0 and n_win % NBUF == 0

    mesh = plsc.VectorSubcoreMesh(core_axis_name="c", subcore_axis_name="s")

    @functools.partial(
        pl.kernel,
        out_type=jax.ShapeDtypeStruct(
            (hist, D_MODEL // 8, batch // W, 8, W), jnp.float32),
        mesh=mesh,
        scratch_types=[
            pltpu.VMEM((hist, cpw * W), jnp.int32),
            pltpu.VMEM((NBUF, W, D_MODEL), jnp.float32),
            pltpu.VMEM((NBUF, D_MODEL // 8, 8, W), jnp.float32),
            pltpu.SemaphoreType.DMA((NBUF,)),
        ],
        compiler_params=pltpu.CompilerParams(
            use_tc_tiling_on_sc=False, needs_layout_passes=False),
    )
    def k(xt_hbm, lut_hbm, out_hbm, idx_v, rows_v, trans_v, gsem):
        wid = lax.axis_index("s") * NC + lax.axis_index("c")
        ct0 = wid * cpw
        pltpu.sync_copy(xt_hbm.at[:, pl.ds(ct0 * W, cpw * W)], idx_v)

        iota = lax.iota(jnp.int32, LANES)

        def fire_gather(w, b):
            h, ct = w // cpw, lax.rem(w, cpw)
            pltpu.async_copy(
                lut_hbm.at[idx_v.at[h, pl.ds(ct * W, W)]],
                rows_v.at[b],
                gsem.at[b],
            )

        def drain_gather(b):
            pltpu.make_async_copy(
                lut_hbm.at[pl.ds(0, W)], rows_v.at[b], gsem.at[b]
            ).wait()

        for b in range(NBUF):
            fire_gather(jnp.int32(b), b)

        @pl.loop(0, n_win, step=NBUF)
        def _(w0):
            for b in range(NBUF):
                w = w0 + b
                drain_gather(b)

                @pl.loop(0, D_MODEL // 8)
                def _(rt):
                    for i in range(8):
                        d = rt * 8 + i
                        dvec = jnp.full((LANES,), d, jnp.int32)
                        for jj in range(W // LANES):
                            v = plsc.load_gather(
                                rows_v.at[b], [iota + (jj * LANES), dvec])
                            trans_v[b, rt, i, pl.ds(jj * LANES, LANES)] = (
                                v * SCALE)

                h, ct = w // cpw, lax.rem(w, cpw)
                pltpu.sync_copy(
                    trans_v.at[b], out_hbm.at[h, :, ct0 + ct, :, :])

                @pl.when(w + NBUF < n_win)
                def _():
                    fire_gather(w + NBUF, b)

    return k(xt, lut)


def kernel(x, lut):
    batch, hist = x.shape
    o5 = _sc_embed(jnp.transpose(x).astype(jnp.int32), lut)
    return o5.transpose(2, 4, 0, 1, 3).reshape(batch, hist, D_MODEL)


# speedup vs baseline: 1.5278x; 1.5278x over previous
import functools
import math

import jax
import jax.numpy as jnp
from jax import lax
from jax.experimental import pallas as pl
from jax.experimental.pallas import tpu as pltpu
from jax.experimental.pallas import tpu_sc as plsc

D_MODEL = 64
SCALE = math.sqrt(D_MODEL)
LANES = 16
NC, NS = 2, 16
NW = NC * NS
W = 128
NBUF = 2


def _sc_embed(xf, lut):
    n_per = xf.shape[1]
    n = NW * n_per
    n_win = n_per // W
    assert n_per % W == 0 and n_win % NBUF == 0

    mesh = plsc.VectorSubcoreMesh(core_axis_name="c", subcore_axis_name="s")

    @functools.partial(
        pl.kernel,
        out_type=jax.ShapeDtypeStruct((n, D_MODEL), jnp.float32),
        mesh=mesh,
        scratch_types=[
            pltpu.VMEM((1, n_per), jnp.int32),
            pltpu.VMEM((NBUF, W, D_MODEL), jnp.float32),
            pltpu.SemaphoreType.DMA((NBUF,)),
        ],
        compiler_params=pltpu.CompilerParams(use_tc_tiling_on_sc=False),
    )
    def k(xf_hbm, lut_hbm, out_hbm, idx_v, rows_v, gsem):
        wid = lax.axis_index("s") * NC + lax.axis_index("c")
        pltpu.sync_copy(xf_hbm.at[pl.ds(wid, 1)], idx_v)
        row0 = wid * n_per

        def fire_gather(w, b):
            pltpu.async_copy(
                lut_hbm.at[idx_v.at[0, pl.ds(w * W, W)]],
                rows_v.at[b],
                gsem.at[b],
            )

        def drain_gather(b):
            pltpu.make_async_copy(
                lut_hbm.at[pl.ds(0, W)], rows_v.at[b], gsem.at[b]
            ).wait()

        for b in range(NBUF):
            fire_gather(jnp.int32(b), b)

        @pl.loop(0, n_win, step=NBUF)
        def _(w0):
            for b in range(NBUF):
                w = w0 + b
                drain_gather(b)

                @pl.loop(0, W)
                def _(r):
                    for c in range(D_MODEL // LANES):
                        sl = pl.ds(c * LANES, LANES)
                        rows_v[b, r, sl] = rows_v[b, r, sl] * SCALE

                pltpu.sync_copy(
                    rows_v.at[b], out_hbm.at[pl.ds(row0 + w * W, W)])

                @pl.when(w + NBUF < n_win)
                def _():
                    fire_gather(w + NBUF, b)

    return k(xf, lut)


def kernel(x, lut):
    batch, hist = x.shape
    xf = x.reshape(NW, (batch * hist) // NW).astype(jnp.int32)
    out = _sc_embed(xf, lut)
    return out.reshape(batch, hist, D_MODEL)


# speedup vs baseline: 1.5548x; 1.0177x over previous
import functools
import math

import jax
import jax.numpy as jnp
from jax import lax
from jax.experimental import pallas as pl
from jax.experimental.pallas import tpu as pltpu
from jax.experimental.pallas import tpu_sc as plsc

D_MODEL = 64
SCALE = math.sqrt(D_MODEL)
LANES = 16
NC, NS = 2, 16
NW = NC * NS
W = 128
NBUF = 2


def _sc_embed(xf, lut):
    n_per = xf.shape[1]
    n = NW * n_per
    n_win = n_per // W
    assert n_per % W == 0 and n_win % NBUF == 0

    mesh = plsc.VectorSubcoreMesh(core_axis_name="c", subcore_axis_name="s")

    @functools.partial(
        pl.kernel,
        out_type=jax.ShapeDtypeStruct((n, D_MODEL), jnp.float32),
        mesh=mesh,
        scratch_types=[
            pltpu.VMEM((1, n_per), jnp.int32),
            pltpu.VMEM((NBUF, W, D_MODEL), jnp.float32),
            pltpu.SemaphoreType.DMA((NBUF,)),
        ],
        compiler_params=pltpu.CompilerParams(use_tc_tiling_on_sc=False),
    )
    def k(xf_hbm, lut_hbm, out_hbm, idx_v, rows_v, gsem):
        wid = lax.axis_index("s") * NC + lax.axis_index("c")
        pltpu.sync_copy(xf_hbm.at[pl.ds(wid, 1)], idx_v)
        row0 = wid * n_per

        def fire_gather(w, b):
            pltpu.async_copy(
                lut_hbm.at[idx_v.at[0, pl.ds(w * W, W)]],
                rows_v.at[b],
                gsem.at[b],
            )

        def drain_gather(b):
            pltpu.make_async_copy(
                lut_hbm.at[pl.ds(0, W)], rows_v.at[b], gsem.at[b]
            ).wait()

        for b in range(NBUF):
            fire_gather(jnp.int32(b), b)

        @pl.loop(0, n_win, step=NBUF)
        def _(w0):
            for b in range(NBUF):
                w = w0 + b
                drain_gather(b)

                @pl.loop(0, W, step=4)
                def _(r0):
                    for dr in range(4):
                        for c in range(D_MODEL // LANES):
                            sl = pl.ds(c * LANES, LANES)
                            rows_v[b, r0 + dr, sl] = (
                                rows_v[b, r0 + dr, sl] * SCALE)

                pltpu.sync_copy(
                    rows_v.at[b], out_hbm.at[pl.ds(row0 + w * W, W)])

                @pl.when(w + NBUF < n_win)
                def _():
                    fire_gather(w + NBUF, b)

    return k(xf, lut)


def kernel(x, lut):
    batch, hist = x.shape
    xf = x.reshape(NW, (batch * hist) // NW).astype(jnp.int32)
    out = _sc_embed(xf, lut)
    return out.reshape(batch, hist, D_MODEL)


# speedup vs baseline: 1.6047x; 1.0321x over previous
import functools
import math

import jax
import jax.numpy as jnp
from jax import lax
from jax.experimental import pallas as pl
from jax.experimental.pallas import tpu as pltpu
from jax.experimental.pallas import tpu_sc as plsc

D_MODEL = 64
SCALE = math.sqrt(D_MODEL)
LANES = 16
NC, NS = 2, 16
NW = NC * NS
W = 256
NBUF = 2


def _sc_embed(xf, lut):
    n_per = xf.shape[1]
    n = NW * n_per
    n_win = n_per // W
    assert n_per % W == 0 and n_win % NBUF == 0

    mesh = plsc.VectorSubcoreMesh(core_axis_name="c", subcore_axis_name="s")

    @functools.partial(
        pl.kernel,
        out_type=jax.ShapeDtypeStruct((n, D_MODEL), jnp.float32),
        mesh=mesh,
        scratch_types=[
            pltpu.VMEM((1, n_per), jnp.int32),
            pltpu.VMEM((NBUF, W, D_MODEL), jnp.float32),
            pltpu.SemaphoreType.DMA((NBUF,)),
        ],
        compiler_params=pltpu.CompilerParams(use_tc_tiling_on_sc=False),
    )
    def k(xf_hbm, lut_hbm, out_hbm, idx_v, rows_v, gsem):
        wid = lax.axis_index("s") * NC + lax.axis_index("c")
        pltpu.sync_copy(xf_hbm.at[pl.ds(wid, 1)], idx_v)
        row0 = wid * n_per

        def fire_gather(w, b):
            pltpu.async_copy(
                lut_hbm.at[idx_v.at[0, pl.ds(w * W, W)]],
                rows_v.at[b],
                gsem.at[b],
            )

        def drain_gather(b):
            pltpu.make_async_copy(
                lut_hbm.at[pl.ds(0, W)], rows_v.at[b], gsem.at[b]
            ).wait()

        for b in range(NBUF):
            fire_gather(jnp.int32(b), b)

        @pl.loop(0, n_win, step=NBUF)
        def _(w0):
            for b in range(NBUF):
                w = w0 + b
                drain_gather(b)

                @pl.loop(0, W, step=4)
                def _(r0):
                    for dr in range(4):
                        for c in range(D_MODEL // LANES):
                            sl = pl.ds(c * LANES, LANES)
                            rows_v[b, r0 + dr, sl] = (
                                rows_v[b, r0 + dr, sl] * SCALE)

                pltpu.sync_copy(
                    rows_v.at[b], out_hbm.at[pl.ds(row0 + w * W, W)])

                @pl.when(w + NBUF < n_win)
                def _():
                    fire_gather(w + NBUF, b)

    return k(xf, lut)


def kernel(x, lut):
    batch, hist = x.shape
    xf = x.reshape(NW, (batch * hist) // NW).astype(jnp.int32)
    out = _sc_embed(xf, lut)
    return out.reshape(batch, hist, D_MODEL)
